# baseline (speedup 1.0000x reference)
import functools

import jax
import jax.numpy as jnp
from jax import lax
from jax.experimental import pallas as pl
from jax.experimental.pallas import tpu as pltpu
from jax.experimental.pallas import tpu_sc as plsc

N = 10000
E = 320000
F = 128
HID = 256
C = 16
NC, NS = 2, 16
NW = NC * NS
NPAD = 10112
ROWS_PER_TILE = NPAD // NS
K = 128
CPW = 80
NCHUNK_PAD = CPW * NW
EPAD = NCHUNK_PAD * K
SUP = 8
NSUP = CPW // SUP
NSLOT = 2
R = 2000

_mesh = plsc.VectorSubcoreMesh(core_axis_name="c", subcore_axis_name="s")


def _make_sc_segsum(gather):
    nslot = NSLOT if gather else 1
    scratch = [
        pltpu.VMEM((SUP, K), jnp.int32),
        pltpu.VMEM((SUP, K), jnp.int32),
        pltpu.VMEM((nslot, K, F), jnp.float32),
        pltpu.VMEM_SHARED((NPAD, F), jnp.float32),
    ] + [pltpu.SemaphoreType.DMA] * (2 * NSLOT)

    @functools.partial(
        pl.kernel,
        out_type=jax.ShapeDtypeStruct((NC * NPAD, F), jnp.float32),
        mesh=_mesh,
        scratch_types=scratch,
    )
    def k(table, pk, zeros, out, sbuf, dbuf, rows_v, acc, *sems):
        gsem = sems[:NSLOT]
        ssem = sems[NSLOT:]
        cid = lax.axis_index("c")
        sid = lax.axis_index("s")
        wid = cid * NS + sid
        row0 = sid * ROWS_PER_TILE
        sl = pl.ds(row0, ROWS_PER_TILE)
        pltpu.sync_copy(zeros.at[sl], acc.at[sl])
        if not gather:
            pltpu.sync_copy(table, rows_v.at[0])
        plsc.subcore_barrier()

        def super_body(t, carry):
            r0 = pl.multiple_of(wid * CPW + t * SUP, SUP)
            pltpu.sync_copy(pk.at[pl.ds(r0, SUP)], sbuf)
            for b in range(SUP):
                for v in range(K // 16):
                    cs = pl.ds(16 * v, 16)
                    p = sbuf[b, cs]
                    dbuf[b, cs] = p >> 16
                    if gather:
                        sbuf[b, cs] = p & 0xFFFF

            if gather:
                def issue_gather(b):
                    s = b % NSLOT
                    return pltpu.async_copy(
                        table.at[sbuf.at[b]], rows_v.at[s], gsem[s])

                def issue_scatter(b):
                    s = b % NSLOT
                    return pltpu.async_copy(
                        rows_v.at[s], acc.at[dbuf.at[b]], ssem[s],
                        add=True)

                pend_s = {}
                gd = {0: issue_gather(0)}
                for b in range(SUP):
                    nb = b + 1
                    if nb < SUP:
                        if nb >= NSLOT:
                            pend_s.pop(nb % NSLOT).wait()
                        gd[nb] = issue_gather(nb)
                    gd.pop(b).wait()
                    pend_s[b % NSLOT] = issue_scatter(b)
                for s in sorted(pend_s):
                    pend_s.pop(s).wait()
            else:
                pend_s = {}
                for b in range(SUP):
                    s = b % NSLOT
                    if b >= NSLOT:
                        pend_s.pop(s).wait()
                    pend_s[s] = pltpu.async_copy(
                        rows_v.at[0], acc.at[dbuf.at[b]], ssem[s],
                        add=True)
                for s in sorted(pend_s):
                    pend_s.pop(s).wait()
            return carry

        lax.fori_loop(0, NSUP, super_body, 0)
        plsc.subcore_barrier()
        out_row = cid * NPAD + row0
        pltpu.sync_copy(acc.at[sl], out.at[pl.ds(out_row, ROWS_PER_TILE)])

    return k


_sc_seg = _make_sc_segsum(gather=True)
_sc_degree = _make_sc_segsum(gather=False)


def _tc_scale_kernel(d0_ref, d1_ref, x_ref, xs_ref, dinv_ref):
    deg = d0_ref[...][:, :C] + d1_ref[...][:, :C] + 1.0
    dinv = lax.rsqrt(deg)
    dinv_ref[...] = dinv
    xs_ref[...] = x_ref[...] * dinv[:, :1]


def _tc_scale(p0, p1, x):
    return pl.pallas_call(
        _tc_scale_kernel,
        grid=(N // R,),
        in_specs=[
            pl.BlockSpec((R, F), lambda i: (i, 0)),
            pl.BlockSpec((R, F), lambda i: (i, 0)),
            pl.BlockSpec((R, F), lambda i: (i, 0)),
        ],
        out_specs=[
            pl.BlockSpec((R, F), lambda i: (i, 0)),
            pl.BlockSpec((R, C), lambda i: (i, 0)),
        ],
        out_shape=[
            jax.ShapeDtypeStruct((N, F), jnp.float32),
            jax.ShapeDtypeStruct((N, C), jnp.float32),
        ],
    )(p0, p1, x)


def _tc_dense_kernel(t0_ref, t1_ref, xs_ref, dinv_ref, W0_ref, b0_ref, W1_ref,
                     zs_ref):
    dinv = dinv_ref[...]
    g = (t0_ref[...] + t1_ref[...] + xs_ref[...]) * dinv[:, :1]
    h = jnp.dot(g, W0_ref[...], preferred_element_type=jnp.float32)
    h = jnp.maximum(h + b0_ref[...], 0.0)
    z = jnp.dot(h, W1_ref[...], preferred_element_type=jnp.float32)
    zs_ref[...] = jnp.concatenate(
        [z * dinv, jnp.zeros((z.shape[0], F - C), jnp.float32)], axis=1)


def _tc_dense(t0, t1, xs, dinv, W0, b0, W1):
    return pl.pallas_call(
        _tc_dense_kernel,
        grid=(N // R,),
        in_specs=[
            pl.BlockSpec((R, F), lambda i: (i, 0)),
            pl.BlockSpec((R, F), lambda i: (i, 0)),
            pl.BlockSpec((R, F), lambda i: (i, 0)),
            pl.BlockSpec((R, C), lambda i: (i, 0)),
            pl.BlockSpec((F, HID), lambda i: (0, 0)),
            pl.BlockSpec((1, HID), lambda i: (0, 0)),
            pl.BlockSpec((HID, C), lambda i: (0, 0)),
        ],
        out_specs=pl.BlockSpec((R, F), lambda i: (i, 0)),
        out_shape=jax.ShapeDtypeStruct((N, F), jnp.float32),
    )(t0, t1, xs, dinv, W0, b0, W1)


def _tc_final_kernel(u0_ref, u1_ref, zs_ref, dinv_ref, b1_ref, out_ref):
    u = (u0_ref[...][:, :C] + u1_ref[...][:, :C] + zs_ref[...][:, :C])
    out_ref[...] = u * dinv_ref[...] + b1_ref[...]


def _tc_final(u0, u1, zs, dinv, b1):
    return pl.pallas_call(
        _tc_final_kernel,
        grid=(N // R,),
        in_specs=[
            pl.BlockSpec((R, F), lambda i: (i, 0)),
            pl.BlockSpec((R, F), lambda i: (i, 0)),
            pl.BlockSpec((R, F), lambda i: (i, 0)),
            pl.BlockSpec((R, C), lambda i: (i, 0)),
            pl.BlockSpec((1, C), lambda i: (0, 0)),
        ],
        out_specs=pl.BlockSpec((R, C), lambda i: (i, 0)),
        out_shape=jax.ShapeDtypeStruct((N, C), jnp.float32),
    )(u0, u1, zs, dinv, b1)


def kernel(x, edge_index, W0, b0, W1, b1):
    pad = EPAD - E
    src = jnp.concatenate([edge_index[0], jnp.zeros((pad,), jnp.int32)])
    dst = jnp.concatenate([edge_index[1],
                           N + (jnp.arange(pad, dtype=jnp.int32)
                                % (NPAD - N))])
    pk = (src | (dst << 16)).reshape(NCHUNK_PAD, K)
    zerosF = jnp.zeros((NPAD, F), jnp.float32)
    onesK = jnp.ones((K, F), jnp.float32)

    dp = _sc_degree(onesK, pk, zerosF)
    xs, dinv = _tc_scale(dp[:N], dp[NPAD:NPAD + N], x)
    tp = _sc_seg(xs, pk, zerosF)
    zs = _tc_dense(tp[:N], tp[NPAD:NPAD + N], xs, dinv, W0,
                   b0.reshape(1, HID), W1)
    zs_pad = jnp.pad(zs, ((0, NPAD - N), (0, 0)))
    up = _sc_seg(zs_pad, pk, zerosF)
    out = _tc_final(up[:N], up[NPAD:NPAD + N], zs, dinv, b1.reshape(1, C))
    return out


# speedup vs baseline: 1.9790x; 1.9790x over previous
import functools

import jax
import jax.numpy as jnp
from jax import lax
from jax.experimental import pallas as pl
from jax.experimental.pallas import tpu as pltpu
from jax.experimental.pallas import tpu_sc as plsc

N = 10000
E = 320000
F = 128
HID = 256
C = 16
NC, NS = 2, 16
NW = NC * NS
NPAD = 10240
ROWS_PER_TILE = NPAD // NS
K = 128
NCHUNKS = E // K
FULL_ROUNDS = NCHUNKS // NW
EXTRA = NCHUNKS - FULL_ROUNDS * NW
R = 2000

_mesh = plsc.VectorSubcoreMesh(core_axis_name="c", subcore_axis_name="s")


def _make_sc_segsum(gather):
    if not gather:
        @functools.partial(
            pl.kernel,
            out_type=jax.ShapeDtypeStruct((NC * NPAD, F), jnp.float32),
            mesh=_mesh,
            scratch_types=[
                pltpu.VMEM((K,), jnp.int32),
                pltpu.VMEM((K,), jnp.int32),
                pltpu.VMEM((K, F), jnp.float32),
                pltpu.VMEM_SHARED((NPAD, F), jnp.float32),
                pltpu.SemaphoreType.DMA,
            ],
        )
        def k(table, src, dst, zeros, out, src_v, dst_v, rows_v, acc, sem):
            cid = lax.axis_index("c")
            sid = lax.axis_index("s")
            wid = cid * NS + sid
            row0 = sid * ROWS_PER_TILE
            sl = pl.ds(row0, ROWS_PER_TILE)
            pltpu.sync_copy(zeros.at[sl], acc.at[sl])
            pltpu.sync_copy(table, rows_v)
            plsc.subcore_barrier()

            n_chunks = FULL_ROUNDS + jnp.where(wid < EXTRA, 1, 0)

            def body(j, carry):
                base = pl.multiple_of((wid + j * NW) * K, K)
                pltpu.sync_copy(dst.at[pl.ds(base, K)], dst_v)
                pltpu.sync_copy(rows_v, acc.at[dst_v], add=True)
                return carry

            lax.fori_loop(0, n_chunks, body, 0)
            plsc.subcore_barrier()
            out_row = cid * NPAD + row0
            pltpu.sync_copy(acc.at[sl], out.at[pl.ds(out_row, ROWS_PER_TILE)])

        return k

    @functools.partial(
        pl.kernel,
        out_type=jax.ShapeDtypeStruct((NC * NPAD, F), jnp.float32),
        mesh=_mesh,
        scratch_types=[
            pltpu.VMEM((K,), jnp.int32),
            pltpu.VMEM((2, K), jnp.int32),
            pltpu.VMEM((2, K, F), jnp.float32),
            pltpu.VMEM_SHARED((NPAD, F), jnp.float32),
            pltpu.SemaphoreType.DMA,
            pltpu.SemaphoreType.DMA,
            pltpu.SemaphoreType.DMA,
        ],
    )
    def k(table, src, dst, zeros, out, src_v, dst_v, rows_v, acc,
          gsem, ssem0, ssem1):
        ssem = (ssem0, ssem1)
        cid = lax.axis_index("c")
        sid = lax.axis_index("s")
        wid = cid * NS + sid
        row0 = sid * ROWS_PER_TILE
        sl = pl.ds(row0, ROWS_PER_TILE)
        pltpu.sync_copy(zeros.at[sl], acc.at[sl])
        plsc.subcore_barrier()

        def chunk(jj, b, defer):
            base = pl.multiple_of((wid + jj * NW) * K, K)
            pltpu.sync_copy(dst.at[pl.ds(base, K)], dst_v.at[b])
            pltpu.sync_copy(src.at[pl.ds(base, K)], src_v)
            pltpu.async_copy(table.at[src_v], rows_v.at[b], gsem).wait()
            d = pltpu.async_copy(rows_v.at[b], acc.at[dst_v.at[b]], ssem[b],
                                 add=True)
            if not defer:
                d.wait()

        def drain(b):
            pltpu.make_async_copy(zeros.at[pl.ds(0, K)], rows_v.at[b],
                                  ssem[b]).wait()

        def body(j, carry):
            for b in range(2):
                @pl.when(j > 0)
                def _():
                    drain(b)
                chunk(2 * j + b, b, defer=True)
            return carry

        lax.fori_loop(0, FULL_ROUNDS // 2, body, 0)
        drain(0)
        drain(1)

        @pl.when(wid < EXTRA)
        def _():
            chunk(FULL_ROUNDS, 0, defer=False)

        plsc.subcore_barrier()
        out_row = cid * NPAD + row0
        pltpu.sync_copy(acc.at[sl], out.at[pl.ds(out_row, ROWS_PER_TILE)])

    return k


_sc_seg = _make_sc_segsum(gather=True)
_sc_degree = _make_sc_segsum(gather=False)


def _tc_scale_kernel(d0_ref, d1_ref, x_ref, xs_ref, dinv_ref):
    deg = d0_ref[...][:, :C] + d1_ref[...][:, :C] + 1.0
    dinv = lax.rsqrt(deg)
    dinv_ref[...] = dinv
    xs_ref[...] = x_ref[...] * dinv[:, :1]


def _tc_scale(p0, p1, x):
    return pl.pallas_call(
        _tc_scale_kernel,
        grid=(N // R,),
        in_specs=[
            pl.BlockSpec((R, F), lambda i: (i, 0)),
            pl.BlockSpec((R, F), lambda i: (i, 0)),
            pl.BlockSpec((R, F), lambda i: (i, 0)),
        ],
        out_specs=[
            pl.BlockSpec((R, F), lambda i: (i, 0)),
            pl.BlockSpec((R, C), lambda i: (i, 0)),
        ],
        out_shape=[
            jax.ShapeDtypeStruct((N, F), jnp.float32),
            jax.ShapeDtypeStruct((N, C), jnp.float32),
        ],
    )(p0, p1, x)


def _tc_dense_kernel(t0_ref, t1_ref, xs_ref, dinv_ref, W0_ref, b0_ref, W1_ref,
                     zs_ref):
    dinv = dinv_ref[...]
    g = (t0_ref[...] + t1_ref[...] + xs_ref[...]) * dinv[:, :1]
    h = jnp.dot(g, W0_ref[...], preferred_element_type=jnp.float32)
    h = jnp.maximum(h + b0_ref[...], 0.0)
    z = jnp.dot(h, W1_ref[...], preferred_element_type=jnp.float32)
    zs_ref[...] = jnp.concatenate(
        [z * dinv, jnp.zeros((z.shape[0], F - C), jnp.float32)], axis=1)


def _tc_dense(t0, t1, xs, dinv, W0, b0, W1):
    return pl.pallas_call(
        _tc_dense_kernel,
        grid=(N // R,),
        in_specs=[
            pl.BlockSpec((R, F), lambda i: (i, 0)),
            pl.BlockSpec((R, F), lambda i: (i, 0)),
            pl.BlockSpec((R, F), lambda i: (i, 0)),
            pl.BlockSpec((R, C), lambda i: (i, 0)),
            pl.BlockSpec((F, HID), lambda i: (0, 0)),
            pl.BlockSpec((1, HID), lambda i: (0, 0)),
            pl.BlockSpec((HID, C), lambda i: (0, 0)),
        ],
        out_specs=pl.BlockSpec((R, F), lambda i: (i, 0)),
        out_shape=jax.ShapeDtypeStruct((N, F), jnp.float32),
    )(t0, t1, xs, dinv, W0, b0, W1)


def _tc_final_kernel(u0_ref, u1_ref, zs_ref, dinv_ref, b1_ref, out_ref):
    u = (u0_ref[...][:, :C] + u1_ref[...][:, :C] + zs_ref[...][:, :C])
    out_ref[...] = u * dinv_ref[...] + b1_ref[...]


def _tc_final(u0, u1, zs, dinv, b1):
    return pl.pallas_call(
        _tc_final_kernel,
        grid=(N // R,),
        in_specs=[
            pl.BlockSpec((R, F), lambda i: (i, 0)),
            pl.BlockSpec((R, F), lambda i: (i, 0)),
            pl.BlockSpec((R, F), lambda i: (i, 0)),
            pl.BlockSpec((R, C), lambda i: (i, 0)),
            pl.BlockSpec((1, C), lambda i: (0, 0)),
        ],
        out_specs=pl.BlockSpec((R, C), lambda i: (i, 0)),
        out_shape=jax.ShapeDtypeStruct((N, C), jnp.float32),
    )(u0, u1, zs, dinv, b1)


def kernel(x, edge_index, W0, b0, W1, b1):
    src = edge_index[0]
    dst = edge_index[1]
    zerosF = jnp.zeros((NPAD, F), jnp.float32)
    onesK = jnp.ones((K, F), jnp.float32)

    dp = _sc_degree(onesK, src, dst, zerosF)
    xs, dinv = _tc_scale(dp[:N], dp[NPAD:NPAD + N], x)
    tp = _sc_seg(xs, src, dst, zerosF)
    zs = _tc_dense(tp[:N], tp[NPAD:NPAD + N], xs, dinv, W0,
                   b0.reshape(1, HID), W1)
    zs_pad = jnp.pad(zs, ((0, NPAD - N), (0, 0)))
    up = _sc_seg(zs_pad, src, dst, zerosF)
    out = _tc_final(up[:N], up[NPAD:NPAD + N], zs, dinv, b1.reshape(1, C))
    return out


# speedup vs baseline: 2.6735x; 1.3510x over previous
import functools

import jax
import jax.numpy as jnp
from jax import lax
from jax.experimental import pallas as pl
from jax.experimental.pallas import tpu as pltpu
from jax.experimental.pallas import tpu_sc as plsc

N = 10000
E = 320000
F = 128
HID = 256
C = 16
NC, NS = 2, 16
NW = NC * NS
NPAD = 10112
ROWS_PER_TILE = NPAD // NS
K = 128
NCHUNKS = E // K
FULL_ROUNDS = NCHUNKS // NW
EXTRA = NCHUNKS - FULL_ROUNDS * NW
R = 2000

_mesh = plsc.VectorSubcoreMesh(core_axis_name="c", subcore_axis_name="s")


def _make_sc_segsum(gather):
    if not gather:
        @functools.partial(
            pl.kernel,
            out_type=jax.ShapeDtypeStruct((NC * NPAD, F), jnp.float32),
            mesh=_mesh,
            scratch_types=[
                pltpu.VMEM((K,), jnp.int32),
                pltpu.VMEM((K,), jnp.int32),
                pltpu.VMEM((K, F), jnp.float32),
                pltpu.VMEM_SHARED((NPAD, F), jnp.float32),
                pltpu.SemaphoreType.DMA,
            ],
        )
        def k(table, src, dst, zeros, out, src_v, dst_v, rows_v, acc, sem):
            cid = lax.axis_index("c")
            sid = lax.axis_index("s")
            wid = cid * NS + sid
            row0 = sid * ROWS_PER_TILE
            sl = pl.ds(row0, ROWS_PER_TILE)
            pltpu.sync_copy(zeros.at[sl], acc.at[sl])
            pltpu.sync_copy(table, rows_v)
            plsc.subcore_barrier()

            n_chunks = FULL_ROUNDS + jnp.where(wid < EXTRA, 1, 0)

            def body(j, carry):
                base = pl.multiple_of((wid + j * NW) * K, K)
                pltpu.sync_copy(dst.at[pl.ds(base, K)], dst_v)
                pltpu.sync_copy(rows_v, acc.at[dst_v], add=True)
                return carry

            lax.fori_loop(0, n_chunks, body, 0)
            plsc.subcore_barrier()
            out_row = cid * NPAD + row0
            pltpu.sync_copy(acc.at[sl], out.at[pl.ds(out_row, ROWS_PER_TILE)])

        return k

    NSL = 3

    @functools.partial(
        pl.kernel,
        out_type=jax.ShapeDtypeStruct((NC * NPAD, F), jnp.float32),
        mesh=_mesh,
        scratch_types=[
            pltpu.VMEM((NSL, K), jnp.int32),
            pltpu.VMEM((NSL, K), jnp.int32),
            pltpu.VMEM((NSL, K, F), jnp.float32),
            pltpu.VMEM_SHARED((NPAD, F), jnp.float32),
        ] + [pltpu.SemaphoreType.DMA] * (2 * NSL),
    )
    def k(table, src, dst, zeros, out, src_v, dst_v, rows_v, acc, *sems):
        gsem = sems[:NSL]
        ssem = sems[NSL:]
        cid = lax.axis_index("c")
        sid = lax.axis_index("s")
        wid = cid * NS + sid
        row0 = sid * ROWS_PER_TILE
        sl = pl.ds(row0, ROWS_PER_TILE)
        pltpu.sync_copy(zeros.at[sl], acc.at[sl])
        plsc.subcore_barrier()

        def stage_and_gather(c, s):
            base = pl.multiple_of((wid + c * NW) * K, K)
            pltpu.sync_copy(dst.at[pl.ds(base, K)], dst_v.at[s])
            pltpu.sync_copy(src.at[pl.ds(base, K)], src_v.at[s])
            pltpu.async_copy(table.at[src_v.at[s]], rows_v.at[s], gsem[s])

        def issue_scatter(s):
            pltpu.async_copy(rows_v.at[s], acc.at[dst_v.at[s]], ssem[s],
                             add=True)

        def drain(sem, s):
            pltpu.make_async_copy(zeros.at[pl.ds(0, K)], rows_v.at[s],
                                  sem).wait()

        def body(j, carry):
            for u in range(NSL):
                @pl.when(j > 0)
                def _():
                    drain(ssem[u], u)
                stage_and_gather(NSL * j + u, u)
                pu = (u - 1) % NSL
                if u == 0:
                    @pl.when(j > 0)
                    def _():
                        drain(gsem[pu], pu)
                        issue_scatter(pu)
                else:
                    drain(gsem[pu], pu)
                    issue_scatter(pu)
            return carry

        lax.fori_loop(0, FULL_ROUNDS // NSL, body, 0)
        last = NSL - 1
        drain(gsem[last], last)
        issue_scatter(last)
        for u in range(NSL):
            drain(ssem[u], u)

        @pl.when(wid < EXTRA)
        def _():
            base = pl.multiple_of((wid + FULL_ROUNDS * NW) * K, K)
            pltpu.sync_copy(dst.at[pl.ds(base, K)], dst_v.at[0])
            pltpu.sync_copy(src.at[pl.ds(base, K)], src_v.at[0])
            pltpu.async_copy(table.at[src_v.at[0]], rows_v.at[0],
                             gsem[0]).wait()
            pltpu.sync_copy(rows_v.at[0], acc.at[dst_v.at[0]], add=True)

        plsc.subcore_barrier()
        out_row = cid * NPAD + row0
        pltpu.sync_copy(acc.at[sl], out.at[pl.ds(out_row, ROWS_PER_TILE)])

    return k


_sc_seg = _make_sc_segsum(gather=True)
_sc_degree = _make_sc_segsum(gather=False)


def _tc_scale_kernel(d0_ref, d1_ref, x_ref, xs_ref, dinv_ref):
    deg = d0_ref[...][:, :C] + d1_ref[...][:, :C] + 1.0
    dinv = lax.rsqrt(deg)
    dinv_ref[...] = dinv
    xs_ref[...] = x_ref[...] * dinv[:, :1]


def _tc_scale(p0, p1, x):
    return pl.pallas_call(
        _tc_scale_kernel,
        grid=(N // R,),
        in_specs=[
            pl.BlockSpec((R, F), lambda i: (i, 0)),
            pl.BlockSpec((R, F), lambda i: (i, 0)),
            pl.BlockSpec((R, F), lambda i: (i, 0)),
        ],
        out_specs=[
            pl.BlockSpec((R, F), lambda i: (i, 0)),
            pl.BlockSpec((R, C), lambda i: (i, 0)),
        ],
        out_shape=[
            jax.ShapeDtypeStruct((N, F), jnp.float32),
            jax.ShapeDtypeStruct((N, C), jnp.float32),
        ],
    )(p0, p1, x)


def _tc_dense_kernel(t0_ref, t1_ref, xs_ref, dinv_ref, W0_ref, b0_ref, W1_ref,
                     zs_ref):
    dinv = dinv_ref[...]
    g = (t0_ref[...] + t1_ref[...] + xs_ref[...]) * dinv[:, :1]
    h = jnp.dot(g, W0_ref[...], preferred_element_type=jnp.float32)
    h = jnp.maximum(h + b0_ref[...], 0.0)
    z = jnp.dot(h, W1_ref[...], preferred_element_type=jnp.float32)
    zs_ref[...] = jnp.concatenate(
        [z * dinv, jnp.zeros((z.shape[0], F - C), jnp.float32)], axis=1)


def _tc_dense(t0, t1, xs, dinv, W0, b0, W1):
    return pl.pallas_call(
        _tc_dense_kernel,
        grid=(N // R,),
        in_specs=[
            pl.BlockSpec((R, F), lambda i: (i, 0)),
            pl.BlockSpec((R, F), lambda i: (i, 0)),
            pl.BlockSpec((R, F), lambda i: (i, 0)),
            pl.BlockSpec((R, C), lambda i: (i, 0)),
            pl.BlockSpec((F, HID), lambda i: (0, 0)),
            pl.BlockSpec((1, HID), lambda i: (0, 0)),
            pl.BlockSpec((HID, C), lambda i: (0, 0)),
        ],
        out_specs=pl.BlockSpec((R, F), lambda i: (i, 0)),
        out_shape=jax.ShapeDtypeStruct((N, F), jnp.float32),
    )(t0, t1, xs, dinv, W0, b0, W1)


def _tc_final_kernel(u0_ref, u1_ref, zs_ref, dinv_ref, b1_ref, out_ref):
    u = (u0_ref[...][:, :C] + u1_ref[...][:, :C] + zs_ref[...][:, :C])
    out_ref[...] = u * dinv_ref[...] + b1_ref[...]


def _tc_final(u0, u1, zs, dinv, b1):
    return pl.pallas_call(
        _tc_final_kernel,
        grid=(N // R,),
        in_specs=[
            pl.BlockSpec((R, F), lambda i: (i, 0)),
            pl.BlockSpec((R, F), lambda i: (i, 0)),
            pl.BlockSpec((R, F), lambda i: (i, 0)),
            pl.BlockSpec((R, C), lambda i: (i, 0)),
            pl.BlockSpec((1, C), lambda i: (0, 0)),
        ],
        out_specs=pl.BlockSpec((R, C), lambda i: (i, 0)),
        out_shape=jax.ShapeDtypeStruct((N, C), jnp.float32),
    )(u0, u1, zs, dinv, b1)


def kernel(x, edge_index, W0, b0, W1, b1):
    src = edge_index[0]
    dst = edge_index[1]
    zerosF = jnp.zeros((NPAD, F), jnp.float32)
    onesK = jnp.ones((K, F), jnp.float32)

    dp = _sc_degree(onesK, src, dst, zerosF)
    xs, dinv = _tc_scale(dp[:N], dp[NPAD:NPAD + N], x)
    tp = _sc_seg(xs, src, dst, zerosF)
    zs = _tc_dense(tp[:N], tp[NPAD:NPAD + N], xs, dinv, W0,
                   b0.reshape(1, HID), W1)
    zs_pad = jnp.pad(zs, ((0, NPAD - N), (0, 0)))
    up = _sc_seg(zs_pad, src, dst, zerosF)
    out = _tc_final(up[:N], up[NPAD:NPAD + N], zs, dinv, b1.reshape(1, C))
    return out


# speedup vs baseline: 2.8939x; 1.0824x over previous
import functools

import jax
import jax.numpy as jnp
from jax import lax
from jax.experimental import pallas as pl
from jax.experimental.pallas import tpu as pltpu
from jax.experimental.pallas import tpu_sc as plsc

N = 10000
E = 320000
F = 128
HID = 256
C = 16
NC, NS = 2, 16
NW = NC * NS
NPAD = 10112
ROWS_PER_TILE = NPAD // NS
K = 128
NCHUNKS = E // K
FULL_ROUNDS = NCHUNKS // NW
EXTRA = NCHUNKS - FULL_ROUNDS * NW
R = 2000

_mesh = plsc.VectorSubcoreMesh(core_axis_name="c", subcore_axis_name="s")


def _make_sc_segsum(gather):
    if not gather:
        @functools.partial(
            pl.kernel,
            out_type=jax.ShapeDtypeStruct((NC * NPAD, F), jnp.float32),
            mesh=_mesh,
            scratch_types=[
                pltpu.VMEM((K,), jnp.int32),
                pltpu.VMEM((2, K), jnp.int32),
                pltpu.VMEM((K, F), jnp.float32),
                pltpu.VMEM_SHARED((NPAD, F), jnp.float32),
                pltpu.SemaphoreType.DMA,
                pltpu.SemaphoreType.DMA,
            ],
        )
        def k(table, src, dst, zeros, out, src_v, dst_v2, rows_v, acc,
              ssem0, ssem1):
            ssem = (ssem0, ssem1)
            cid = lax.axis_index("c")
            sid = lax.axis_index("s")
            wid = cid * NS + sid
            row0 = sid * ROWS_PER_TILE
            sl = pl.ds(row0, ROWS_PER_TILE)
            pltpu.sync_copy(zeros.at[sl], acc.at[sl])
            pltpu.sync_copy(table, rows_v)
            plsc.subcore_barrier()

            def drain(b):
                pltpu.make_async_copy(zeros.at[pl.ds(0, K)], rows_v,
                                      ssem[b]).wait()

            def body(j, carry):
                for b in range(2):
                    @pl.when(j > 0)
                    def _():
                        drain(b)
                    base = pl.multiple_of((wid + (2 * j + b) * NW) * K, K)
                    pltpu.sync_copy(dst.at[pl.ds(base, K)], dst_v2.at[b])
                    pltpu.async_copy(rows_v, acc.at[dst_v2.at[b]], ssem[b],
                                     add=True)
                return carry

            lax.fori_loop(0, FULL_ROUNDS // 2, body, 0)
            drain(0)
            drain(1)

            @pl.when(wid < EXTRA)
            def _():
                base = pl.multiple_of((wid + FULL_ROUNDS * NW) * K, K)
                pltpu.sync_copy(dst.at[pl.ds(base, K)], dst_v2.at[0])
                pltpu.sync_copy(rows_v, acc.at[dst_v2.at[0]], add=True)

            plsc.subcore_barrier()
            out_row = cid * NPAD + row0
            pltpu.sync_copy(acc.at[sl], out.at[pl.ds(out_row, ROWS_PER_TILE)])

        return k

    NSL = 3

    @functools.partial(
        pl.kernel,
        out_type=jax.ShapeDtypeStruct((NC * NPAD, F), jnp.float32),
        mesh=_mesh,
        scratch_types=[
            pltpu.VMEM((NSL, K), jnp.int32),
            pltpu.VMEM((NSL, K), jnp.int32),
            pltpu.VMEM((NSL, K, F), jnp.float32),
            pltpu.VMEM_SHARED((NPAD, F), jnp.float32),
        ] + [pltpu.SemaphoreType.DMA] * (2 * NSL),
    )
    def k(table, src, dst, zeros, out, src_v, dst_v, rows_v, acc, *sems):
        gsem = sems[:NSL]
        ssem = sems[NSL:]
        cid = lax.axis_index("c")
        sid = lax.axis_index("s")
        wid = cid * NS + sid
        row0 = sid * ROWS_PER_TILE
        sl = pl.ds(row0, ROWS_PER_TILE)
        pltpu.sync_copy(zeros.at[sl], acc.at[sl])
        plsc.subcore_barrier()

        def stage_and_gather(c, s):
            base = pl.multiple_of((wid + c * NW) * K, K)
            pltpu.sync_copy(dst.at[pl.ds(base, K)], dst_v.at[s])
            pltpu.sync_copy(src.at[pl.ds(base, K)], src_v.at[s])
            pltpu.async_copy(table.at[src_v.at[s]], rows_v.at[s], gsem[s])

        def issue_scatter(s):
            pltpu.async_copy(rows_v.at[s], acc.at[dst_v.at[s]], ssem[s],
                             add=True)

        def drain(sem, s):
            pltpu.make_async_copy(zeros.at[pl.ds(0, K)], rows_v.at[s],
                                  sem).wait()

        def body(j, carry):
            for u in range(NSL):
                @pl.when(j > 0)
                def _():
                    drain(ssem[u], u)
                stage_and_gather(NSL * j + u, u)
                pu = (u - 1) % NSL
                if u == 0:
                    @pl.when(j > 0)
                    def _():
                        drain(gsem[pu], pu)
                        issue_scatter(pu)
                else:
                    drain(gsem[pu], pu)
                    issue_scatter(pu)
            return carry

        lax.fori_loop(0, FULL_ROUNDS // NSL, body, 0)
        last = NSL - 1
        drain(gsem[last], last)
        issue_scatter(last)
        for u in range(NSL):
            drain(ssem[u], u)

        @pl.when(wid < EXTRA)
        def _():
            base = pl.multiple_of((wid + FULL_ROUNDS * NW) * K, K)
            pltpu.sync_copy(dst.at[pl.ds(base, K)], dst_v.at[0])
            pltpu.sync_copy(src.at[pl.ds(base, K)], src_v.at[0])
            pltpu.async_copy(table.at[src_v.at[0]], rows_v.at[0],
                             gsem[0]).wait()
            pltpu.sync_copy(rows_v.at[0], acc.at[dst_v.at[0]], add=True)

        plsc.subcore_barrier()
        out_row = cid * NPAD + row0
        pltpu.sync_copy(acc.at[sl], out.at[pl.ds(out_row, ROWS_PER_TILE)])

    return k


_sc_seg = _make_sc_segsum(gather=True)
_sc_degree = _make_sc_segsum(gather=False)


def _tc_scale_kernel(d0_ref, d1_ref, x_ref, xs_ref, dinv_ref):
    deg = d0_ref[...][:, :C] + d1_ref[...][:, :C] + 1.0
    dinv = lax.rsqrt(deg)
    dinv_ref[...] = dinv
    xs_ref[...] = x_ref[...] * dinv[:, :1]


def _tc_scale(p0, p1, x):
    return pl.pallas_call(
        _tc_scale_kernel,
        grid=(N // R,),
        in_specs=[
            pl.BlockSpec((R, F), lambda i: (i, 0)),
            pl.BlockSpec((R, F), lambda i: (i, 0)),
            pl.BlockSpec((R, F), lambda i: (i, 0)),
        ],
        out_specs=[
            pl.BlockSpec((R, F), lambda i: (i, 0)),
            pl.BlockSpec((R, C), lambda i: (i, 0)),
        ],
        out_shape=[
            jax.ShapeDtypeStruct((N, F), jnp.float32),
            jax.ShapeDtypeStruct((N, C), jnp.float32),
        ],
    )(p0, p1, x)


def _tc_dense_kernel(t0_ref, t1_ref, xs_ref, dinv_ref, W0_ref, b0_ref, W1_ref,
                     zs_ref):
    dinv = dinv_ref[...]
    g = (t0_ref[...] + t1_ref[...] + xs_ref[...]) * dinv[:, :1]
    h = jnp.dot(g, W0_ref[...], preferred_element_type=jnp.float32)
    h = jnp.maximum(h + b0_ref[...], 0.0)
    z = jnp.dot(h, W1_ref[...], preferred_element_type=jnp.float32)
    zs_ref[...] = jnp.concatenate(
        [z * dinv, jnp.zeros((z.shape[0], F - C), jnp.float32)], axis=1)


def _tc_dense(t0, t1, xs, dinv, W0, b0, W1):
    return pl.pallas_call(
        _tc_dense_kernel,
        grid=(N // R,),
        in_specs=[
            pl.BlockSpec((R, F), lambda i: (i, 0)),
            pl.BlockSpec((R, F), lambda i: (i, 0)),
            pl.BlockSpec((R, F), lambda i: (i, 0)),
            pl.BlockSpec((R, C), lambda i: (i, 0)),
            pl.BlockSpec((F, HID), lambda i: (0, 0)),
            pl.BlockSpec((1, HID), lambda i: (0, 0)),
            pl.BlockSpec((HID, C), lambda i: (0, 0)),
        ],
        out_specs=pl.BlockSpec((R, F), lambda i: (i, 0)),
        out_shape=jax.ShapeDtypeStruct((N, F), jnp.float32),
    )(t0, t1, xs, dinv, W0, b0, W1)


def _tc_final_kernel(u0_ref, u1_ref, zs_ref, dinv_ref, b1_ref, out_ref):
    u = (u0_ref[...][:, :C] + u1_ref[...][:, :C] + zs_ref[...][:, :C])
    out_ref[...] = u * dinv_ref[...] + b1_ref[...]


def _tc_final(u0, u1, zs, dinv, b1):
    return pl.pallas_call(
        _tc_final_kernel,
        grid=(N // R,),
        in_specs=[
            pl.BlockSpec((R, F), lambda i: (i, 0)),
            pl.BlockSpec((R, F), lambda i: (i, 0)),
            pl.BlockSpec((R, F), lambda i: (i, 0)),
            pl.BlockSpec((R, C), lambda i: (i, 0)),
            pl.BlockSpec((1, C), lambda i: (0, 0)),
        ],
        out_specs=pl.BlockSpec((R, C), lambda i: (i, 0)),
        out_shape=jax.ShapeDtypeStruct((N, C), jnp.float32),
    )(u0, u1, zs, dinv, b1)


def kernel(x, edge_index, W0, b0, W1, b1):
    src = edge_index[0]
    dst = edge_index[1]
    zerosF = jnp.zeros((NPAD, F), jnp.float32)
    onesK = jnp.ones((K, F), jnp.float32)

    dp = _sc_degree(onesK, src, dst, zerosF)
    xs, dinv = _tc_scale(dp[:N], dp[NPAD:NPAD + N], x)
    tp = _sc_seg(xs, src, dst, zerosF)
    zs = _tc_dense(tp[:N], tp[NPAD:NPAD + N], xs, dinv, W0,
                   b0.reshape(1, HID), W1)
    zs_pad = jnp.pad(zs, ((0, NPAD - N), (0, 0)))
    up = _sc_seg(zs_pad, src, dst, zerosF)
    out = _tc_final(up[:N], up[NPAD:NPAD + N], zs, dinv, b1.reshape(1, C))
    return out
